# baseline (speedup 1.0000x reference)
import functools

import jax
import jax.numpy as jnp
from jax import lax
from jax.experimental import pallas as pl
from jax.experimental.pallas import tpu as pltpu
from jax.experimental.pallas import tpu_sc as plsc

_NUM_CORES = 2
_NUM_SUBCORES = 16
_NW = _NUM_CORES * _NUM_SUBCORES


def kernel(tokens, W_pos):
    batch, seq = tokens.shape
    d = W_pos.shape[1]
    rows_per_w = seq // _NW
    chunk = 16
    n_chunks = rows_per_w // chunk

    mesh = plsc.VectorSubcoreMesh(core_axis_name="c", subcore_axis_name="s")

    @functools.partial(
        pl.kernel,
        mesh=mesh,
        out_type=jax.ShapeDtypeStruct((batch, seq, d), W_pos.dtype),
        scratch_types=[
            pltpu.VMEM((2, chunk, d), W_pos.dtype),
            pltpu.SemaphoreType.DMA,
            pltpu.SemaphoreType.DMA,
        ],
    )
    def _copy(w_hbm, out_hbm, buf, sem_in, sem_out):
        wid = lax.axis_index("s") * _NUM_CORES + lax.axis_index("c")
        base = wid * rows_per_w
        ins = [None] * n_chunks
        outs = [None] * n_chunks
        ins[0] = pltpu.async_copy(
            w_hbm.at[pl.ds(base, chunk), :], buf.at[0], sem_in)
        for ci in range(n_chunks):
            cur = ci % 2
            ins[ci].wait()
            if ci >= 1:
                for h in outs[ci - 1]:
                    h.wait()
            if ci + 1 < n_chunks:
                nstart = base + (ci + 1) * chunk
                ins[ci + 1] = pltpu.async_copy(
                    w_hbm.at[pl.ds(nstart, chunk), :], buf.at[1 - cur], sem_in)
            start = base + ci * chunk
            outs[ci] = [
                pltpu.async_copy(
                    buf.at[cur], out_hbm.at[b, pl.ds(start, chunk), :], sem_out)
                for b in range(batch)
            ]
        for h in outs[n_chunks - 1]:
            h.wait()

    return _copy(W_pos)
